# Initial kernel scaffold; baseline (speedup 1.0000x reference)
import jax
import jax.numpy as jnp
from jax.experimental import pallas as pl


def kernel(x, edge_index, batch, Wl1, bl1, Wr1, Wl2, bl2, Wr2, Wl3, bl3, Wr3, Wl4, bl4, Wr4):
    raise NotImplementedError("write your pallas kernel here")


# speedup vs baseline: 11.9523x; 11.9523x over previous
import functools
import math

import jax
import jax.numpy as jnp
import numpy as np
from jax import lax
from jax.experimental import pallas as pl
from jax.experimental.pallas import tpu as pltpu
from jax.experimental.pallas import tpu_sc as plsc

N = 100000
E = 3200000
F_IN = 14
H = 64
C = 2
G = 512

NC = 2
NS = 16
NW = NC * NS
LANES = 128
B = 1024
JROWS = B // LANES

EPAD = math.ceil(E / (NW * B)) * (NW * B)
NBLK = EPAD // (NW * B)
NAPAD = 100352
EPOOL = math.ceil(N / (NW * B)) * (NW * B)
NA_POOL = 2048
NB = 1000
NGRID = N // NB


def _make_seg16(nepad, napad):
    nblk = nepad // (NW * B)
    za = napad // NS

    @functools.partial(
        pl.kernel,
        out_type=jax.ShapeDtypeStruct((NC, napad, 16), jnp.float32),
        mesh=plsc.VectorSubcoreMesh(core_axis_name="c", subcore_axis_name="s"),
        compiler_params=pltpu.CompilerParams(use_tc_tiling_on_sc=False),
        scratch_types=[
            pltpu.VMEM((JROWS, LANES), jnp.int32),
            pltpu.VMEM((JROWS, LANES), jnp.int32),
            pltpu.VMEM((B, 16), jnp.float32),
            pltpu.VMEM((LANES, 16), jnp.float32),
            pltpu.SemaphoreType.DMA,
            pltpu.VMEM_SHARED((napad, 16), jnp.float32),
        ],
    )
    def seg(tab, srcr, dstr, out, srcv, dstv, rows, zbuf, sem, acc):
        c = lax.axis_index("c")
        s = lax.axis_index("s")
        w = s * NC + c

        def zb(m, carry):
            zbuf[m, :] = jnp.zeros((16,), jnp.float32)
            return carry

        lax.fori_loop(0, LANES, zb, 0)

        def zc(k, carry):
            pltpu.sync_copy(zbuf, acc.at[pl.ds(s * za + k * LANES, LANES)])
            return carry

        lax.fori_loop(0, za // LANES, zc, 0)
        plsc.subcore_barrier()

        def blk(i, carry):
            r0 = (w * nblk + i) * JROWS
            pltpu.sync_copy(srcr.at[pl.ds(r0, JROWS)], srcv)
            pltpu.sync_copy(dstr.at[pl.ds(r0, JROWS)], dstv)
            handles = [
                pltpu.async_copy(
                    tab.at[srcv.at[j]], rows.at[pl.ds(j * LANES, LANES)], sem
                )
                for j in range(JROWS)
            ]
            for h_ in handles:
                h_.wait()
            for j in range(JROWS):
                pltpu.sync_copy(
                    rows.at[pl.ds(j * LANES, LANES)],
                    acc.at[dstv.at[j]],
                    add=True,
                )
            return carry

        lax.fori_loop(0, nblk, blk, 0)
        plsc.subcore_barrier()
        pltpu.sync_copy(acc.at[pl.ds(s * za, za)], out.at[c, pl.ds(s * za, za)])

    return seg


_seg_main = _make_seg16(EPAD, NAPAD)
_seg_pool = _make_seg16(EPOOL, NA_POOL)


def _dense1_body(x_ref, p_ref, wl_ref, wr_ref, b_ref, hg_ref, rc_ref):
    agg = p_ref[0] + p_ref[1]
    cnt = agg[:, 14]
    rc = 1.0 / jnp.maximum(cnt, 1.0)
    mean = agg * rc[:, None]
    h = jnp.dot(mean, wl_ref[...], preferred_element_type=jnp.float32)
    h = h + jnp.dot(x_ref[...], wr_ref[...], preferred_element_type=jnp.float32)
    h = jnp.maximum(h + b_ref[...], 0.0)
    for g in range(4):
        hg_ref[g] = h[:, g * 16:(g + 1) * 16]
    rc_ref[...] = jnp.broadcast_to(rc[:, None], (NB, 16))


def _dense4_body(p_ref, z_ref, rc_ref, b_ref, sh_ref, out_ref):
    s = p_ref[0] + p_ref[1]
    t = s * rc_ref[...]
    col = lax.broadcasted_iota(jnp.int32, (NB, 16), 1)
    m01 = (col < 2).astype(jnp.float32)
    zsh = jnp.dot(z_ref[...], sh_ref[...], preferred_element_type=jnp.float32)
    out_ref[...] = t * m01 + b_ref[...] + zsh + (col == 2).astype(jnp.float32)


def _final_body(p_ref, out_ref):
    s = p_ref[0] + p_ref[1]
    cnt = s[:, 2]
    rcg = 1.0 / jnp.maximum(cnt, 1.0)
    col = lax.broadcasted_iota(jnp.int32, (G, 16), 1)
    pooled = jnp.where(col < 2, s * rcg[:, None], -1e30)
    m = jnp.max(pooled, axis=1, keepdims=True)
    lse = m + jnp.log(jnp.sum(jnp.exp(pooled - m), axis=1, keepdims=True))
    out_ref[...] = (pooled - lse)[:, 0:2]


def _pad_edges(src, dst, nepad, trash_base):
    npad = nepad - src.shape[0]
    fill = jnp.arange(npad, dtype=jnp.int32) % 8
    srcp = jnp.concatenate([src, fill])
    dstp = jnp.concatenate([dst, trash_base + fill])
    return srcp.reshape(-1, LANES), dstp.reshape(-1, LANES)


def kernel(x, edge_index, batch, Wl1, bl1, Wr1, Wl2, bl2, Wr2, Wl3, bl3, Wr3,
           Wl4, bl4, Wr4):
    f32 = jnp.float32
    src = edge_index[0]
    dst = edge_index[1]
    src2d, dst2d = _pad_edges(src, dst, EPAD, N)
    psrc2d, pdst2d = _pad_edges(
        jnp.arange(N, dtype=jnp.int32), batch, EPOOL, G)

    ones_col = jnp.ones((N, 1), f32)
    zeros_col = jnp.zeros((N, 1), f32)
    x_pad = jnp.concatenate([x, ones_col, zeros_col], axis=1)

    wl1p = jnp.pad(Wl1.T, ((0, 2), (0, 0)))
    wr1p = jnp.pad(Wr1.T, ((0, 2), (0, 0)))
    wl2g = Wl2.T.reshape(4, 16, H)
    wr2g = Wr2.T.reshape(4, 16, H)
    wl3g = Wl3.T.reshape(4, 16, H)
    wr3g = Wr3.T.reshape(4, 16, H)
    w4cat = jnp.pad(
        jnp.concatenate([Wl4.T, Wr4.T], axis=1), ((0, 0), (0, 12)))
    bl1r = bl1.reshape(1, H)
    bl2r = bl2.reshape(1, H)
    bl3r = bl3.reshape(1, H)
    bl4p = jnp.pad(bl4, (0, 14)).reshape(1, 16)
    sh4 = jnp.asarray(np.eye(16, k=-2, dtype=np.float32))

    wspec = pl.BlockSpec((16, H), lambda i: (0, 0))
    wgspec = pl.BlockSpec((4, 16, H), lambda i: (0, 0, 0))
    bspec = pl.BlockSpec((1, H), lambda i: (0, 0))
    b16spec = pl.BlockSpec((1, 16), lambda i: (0, 0))
    nb16 = pl.BlockSpec((NB, 16), lambda i: (i, 0))
    hgspec = pl.BlockSpec((4, NB, 16), lambda i: (0, i, 0))
    pspec = pl.BlockSpec((2, NB, 16), lambda i: (0, i, 0))
    pgspec = pl.BlockSpec((4, 2, NB, 16), lambda i: (0, 0, i, 0))
    shspec = pl.BlockSpec((16, 16), lambda i: (0, 0))

    p1 = _seg_main(x_pad, src2d, dst2d)
    hg1, rc = pl.pallas_call(
        _dense1_body,
        grid=(NGRID,),
        in_specs=[nb16, pspec, wspec, wspec, bspec],
        out_specs=[hgspec, nb16],
        out_shape=[
            jax.ShapeDtypeStruct((4, N, 16), f32),
            jax.ShapeDtypeStruct((N, 16), f32),
        ],
    )(x_pad, p1, wl1p, wr1p, bl1r)

    def conv_mid(hg, wlg, wrg, blr, with_z):
        parts = jnp.stack([_seg_main(hg[g], src2d, dst2d) for g in range(4)])
        outs = [jax.ShapeDtypeStruct((4, N, 16), f32)]
        out_specs = [hgspec]
        if with_z:
            outs.append(jax.ShapeDtypeStruct((N, 16), f32))
            out_specs.append(nb16)
        body = _dense23z_body if with_z else _dense23n_body
        in_specs = [hgspec, pgspec, nb16, wgspec, wgspec, bspec]
        args = [hg, parts, rc, wlg, wrg, blr]
        if with_z:
            in_specs.append(shspec_w4)
            args.append(w4cat)
        return pl.pallas_call(
            body, grid=(NGRID,), in_specs=in_specs,
            out_specs=out_specs, out_shape=outs,
        )(*args)

    shspec_w4 = pl.BlockSpec((H, 16), lambda i: (0, 0))
    hg2 = conv_mid(hg1, wl2g, wr2g, bl2r, False)[0]
    hg3, z4 = conv_mid(hg2, wl3g, wr3g, bl3r, True)

    p4 = _seg_main(z4, src2d, dst2d)
    pp = pl.pallas_call(
        _dense4_body,
        grid=(NGRID,),
        in_specs=[pspec, nb16, nb16, b16spec, shspec],
        out_specs=nb16,
        out_shape=jax.ShapeDtypeStruct((N, 16), f32),
    )(p4, z4, rc, bl4p, sh4)

    ppart = _seg_pool(pp, psrc2d, pdst2d)
    out = pl.pallas_call(
        _final_body,
        grid=(1,),
        in_specs=[pl.BlockSpec((2, G, 16), lambda i: (0, 0, 0))],
        out_specs=pl.BlockSpec((G, C), lambda i: (0, 0)),
        out_shape=jax.ShapeDtypeStruct((G, C), f32),
    )(ppart)
    return out


def _dense23n_body(hgb_ref, p_ref, rc_ref, wl_ref, wr_ref, b_ref, hg_out):
    acc = jnp.broadcast_to(b_ref[...], (NB, H))
    rc = rc_ref[...]
    for g in range(4):
        mean_g = (p_ref[g, 0] + p_ref[g, 1]) * rc
        acc = acc + jnp.dot(mean_g, wl_ref[g], preferred_element_type=jnp.float32)
        acc = acc + jnp.dot(hgb_ref[g], wr_ref[g], preferred_element_type=jnp.float32)
    h = jnp.maximum(acc, 0.0)
    for g in range(4):
        hg_out[g] = h[:, g * 16:(g + 1) * 16]


def _dense23z_body(hgb_ref, p_ref, rc_ref, wl_ref, wr_ref, b_ref, w4_ref,
                   hg_out, z_out):
    acc = jnp.broadcast_to(b_ref[...], (NB, H))
    rc = rc_ref[...]
    for g in range(4):
        mean_g = (p_ref[g, 0] + p_ref[g, 1]) * rc
        acc = acc + jnp.dot(mean_g, wl_ref[g], preferred_element_type=jnp.float32)
        acc = acc + jnp.dot(hgb_ref[g], wr_ref[g], preferred_element_type=jnp.float32)
    h = jnp.maximum(acc, 0.0)
    for g in range(4):
        hg_out[g] = h[:, g * 16:(g + 1) * 16]
    z_out[...] = jnp.dot(h, w4_ref[...], preferred_element_type=jnp.float32)


# speedup vs baseline: 16.8418x; 1.4091x over previous
import functools
import math

import jax
import jax.numpy as jnp
import numpy as np
from jax import lax
from jax.experimental import pallas as pl
from jax.experimental.pallas import tpu as pltpu
from jax.experimental.pallas import tpu_sc as plsc

N = 100000
E = 3200000
F_IN = 14
H = 64
C = 2
G = 512

NC = 2
NS = 16
NW = NC * NS
LANES = 128
B = 512
JROWS = B // LANES

_EBLK = NW * B
EPAD = math.ceil(E / (2 * _EBLK)) * (2 * _EBLK)
NBLK = EPAD // _EBLK
NAPAD = 100352
EPOOL = math.ceil(N / (2 * _EBLK)) * (2 * _EBLK)
NA_POOL = 2048
NB = 1000
NGRID = N // NB


def _make_seg16(nepad, napad, linear_payload=False):
    nblk = nepad // _EBLK
    assert nblk % 2 == 0
    nsteps = nblk // 2
    za = napad // NS

    @functools.partial(
        pl.kernel,
        out_type=jax.ShapeDtypeStruct((NC, napad, 16), jnp.float32),
        mesh=plsc.VectorSubcoreMesh(core_axis_name="c", subcore_axis_name="s"),
        compiler_params=pltpu.CompilerParams(use_tc_tiling_on_sc=False),
        scratch_types=[
            pltpu.VMEM((JROWS, LANES), jnp.int32),
            pltpu.VMEM((JROWS, LANES), jnp.int32),
            pltpu.VMEM((JROWS, LANES), jnp.int32),
            pltpu.VMEM((JROWS, LANES), jnp.int32),
            pltpu.VMEM((B, 16), jnp.float32),
            pltpu.VMEM((B, 16), jnp.float32),
            pltpu.SemaphoreType.DMA,
            pltpu.SemaphoreType.DMA,
            pltpu.SemaphoreType.DMA,
            pltpu.SemaphoreType.DMA,
            pltpu.SemaphoreType.DMA,
            pltpu.SemaphoreType.DMA,
            pltpu.VMEM_SHARED((napad, 16), jnp.float32),
        ],
    )
    def seg(tab, srcr, dstr, out, sv0, dv0, sv1, dv1, rows0, rows1,
            semi0, semi1, semg0, semg1, sems0, sems1, acc):
        c = lax.axis_index("c")
        s = lax.axis_index("s")
        w = s * NC + c
        base = w * nblk

        sv = (sv0, sv1)
        dv = (dv0, dv1)
        rows = (rows0, rows1)
        semi = (semi0, semi1)
        semg = (semg0, semg1)
        sems = (sems0, sems1)

        def idx_issue(blk_i, b):
            r0 = (base + blk_i) * JROWS
            pltpu.async_copy(srcr.at[pl.ds(r0, JROWS)], sv[b], semi[b])
            pltpu.async_copy(dstr.at[pl.ds(r0, JROWS)], dv[b], semi[b])

        def idx_wait(b):
            pltpu.make_async_copy(srcr.at[pl.ds(0, JROWS)], sv[b], semi[b]).wait()
            pltpu.make_async_copy(dstr.at[pl.ds(0, JROWS)], dv[b], semi[b]).wait()

        def gather_issue(blk_i, b):
            if linear_payload:
                r0 = (base + blk_i) * B
                pltpu.async_copy(tab.at[pl.ds(r0, B)], rows[b], semg[b])
            else:
                for j in range(JROWS):
                    pltpu.async_copy(
                        tab.at[sv[b].at[j]],
                        rows[b].at[pl.ds(j * LANES, LANES)],
                        semg[b],
                    )

        def gather_wait(b):
            if linear_payload:
                pltpu.make_async_copy(
                    tab.at[pl.ds(0, B)], rows[b], semg[b]).wait()
            else:
                for j in range(JROWS):
                    pltpu.make_async_copy(
                        tab.at[sv[b].at[j]],
                        rows[b].at[pl.ds(j * LANES, LANES)],
                        semg[b],
                    ).wait()

        def scatter_issue(b):
            for j in range(JROWS):
                pltpu.async_copy(
                    rows[b].at[pl.ds(j * LANES, LANES)],
                    acc.at[dv[b].at[j]],
                    sems[b],
                    add=True,
                )

        def scatter_wait(b):
            for j in range(JROWS):
                pltpu.make_async_copy(
                    rows[b].at[pl.ds(j * LANES, LANES)],
                    acc.at[dv[b].at[j]],
                    sems[b],
                ).wait()

        def zb(m, carry):
            rows0[m, :] = jnp.zeros((16,), jnp.float32)
            return carry

        lax.fori_loop(0, LANES, zb, 0)

        def zc(k, carry):
            pltpu.sync_copy(rows0.at[pl.ds(0, LANES)],
                            acc.at[pl.ds(s * za + k * LANES, LANES)])
            return carry

        lax.fori_loop(0, za // LANES, zc, 0)
        plsc.subcore_barrier()

        idx_issue(0, 0)
        idx_issue(1, 1)
        idx_wait(0)
        gather_issue(0, 0)
        idx_wait(1)
        gather_issue(1, 1)
        gather_wait(0)
        idx_issue(jnp.minimum(2, nblk - 1), 0)
        scatter_issue(0)
        gather_wait(1)
        idx_issue(jnp.minimum(3, nblk - 1), 1)
        scatter_issue(1)

        def step(t, carry):
            idx_wait(0)
            scatter_wait(0)
            gather_issue(2 * t, 0)
            idx_wait(1)
            scatter_wait(1)
            gather_issue(2 * t + 1, 1)
            gather_wait(0)
            idx_issue(jnp.minimum(2 * t + 2, nblk - 1), 0)
            scatter_issue(0)
            gather_wait(1)
            idx_issue(jnp.minimum(2 * t + 3, nblk - 1), 1)
            scatter_issue(1)
            return carry

        lax.fori_loop(1, nsteps, step, 0)

        idx_wait(0)
        idx_wait(1)
        scatter_wait(0)
        scatter_wait(1)

        plsc.subcore_barrier()
        pltpu.sync_copy(acc.at[pl.ds(s * za, za)], out.at[c, pl.ds(s * za, za)])

    return seg


_seg_main = _make_seg16(EPAD, NAPAD)
_seg_pool = _make_seg16(EPOOL, NA_POOL, linear_payload=True)


def _dense1_body(x_ref, p_ref, wl_ref, wr_ref, b_ref, hg_ref, rc_ref):
    agg = p_ref[0] + p_ref[1]
    cnt = agg[:, 14]
    rc = 1.0 / jnp.maximum(cnt, 1.0)
    mean = agg * rc[:, None]
    h = jnp.dot(mean, wl_ref[...], preferred_element_type=jnp.float32)
    h = h + jnp.dot(x_ref[...], wr_ref[...], preferred_element_type=jnp.float32)
    h = jnp.maximum(h + b_ref[...], 0.0)
    for g in range(4):
        hg_ref[g] = h[:, g * 16:(g + 1) * 16]
    rc_ref[...] = jnp.broadcast_to(rc[:, None], (NB, 16))


def _dense4_body(p_ref, z_ref, rc_ref, b_ref, sh_ref, out_ref):
    s = p_ref[0] + p_ref[1]
    t = s * rc_ref[...]
    col = lax.broadcasted_iota(jnp.int32, (NB, 16), 1)
    m01 = (col < 2).astype(jnp.float32)
    zsh = jnp.dot(z_ref[...], sh_ref[...], preferred_element_type=jnp.float32)
    out_ref[...] = t * m01 + b_ref[...] + zsh + (col == 2).astype(jnp.float32)


def _final_body(p_ref, out_ref):
    s = p_ref[0] + p_ref[1]
    cnt = s[:, 2]
    rcg = 1.0 / jnp.maximum(cnt, 1.0)
    col = lax.broadcasted_iota(jnp.int32, (G, 16), 1)
    pooled = jnp.where(col < 2, s * rcg[:, None], -1e30)
    m = jnp.max(pooled, axis=1, keepdims=True)
    lse = m + jnp.log(jnp.sum(jnp.exp(pooled - m), axis=1, keepdims=True))
    out_ref[...] = (pooled - lse)[:, 0:2]


def _pad_edges(src, dst, nepad, trash_base):
    npad = nepad - src.shape[0]
    fill = jnp.arange(npad, dtype=jnp.int32) % 8
    srcp = jnp.concatenate([src, fill])
    dstp = jnp.concatenate([dst, trash_base + fill])
    return srcp.reshape(-1, LANES), dstp.reshape(-1, LANES)


def kernel(x, edge_index, batch, Wl1, bl1, Wr1, Wl2, bl2, Wr2, Wl3, bl3, Wr3,
           Wl4, bl4, Wr4):
    f32 = jnp.float32
    src = edge_index[0]
    dst = edge_index[1]
    src2d, dst2d = _pad_edges(src, dst, EPAD, N)
    psrc2d, pdst2d = _pad_edges(
        jnp.arange(N, dtype=jnp.int32), batch, EPOOL, G)

    ones_col = jnp.ones((N, 1), f32)
    zeros_col = jnp.zeros((N, 1), f32)
    x_pad = jnp.concatenate([x, ones_col, zeros_col], axis=1)

    wl1p = jnp.pad(Wl1.T, ((0, 2), (0, 0)))
    wr1p = jnp.pad(Wr1.T, ((0, 2), (0, 0)))
    wl2g = Wl2.T.reshape(4, 16, H)
    wr2g = Wr2.T.reshape(4, 16, H)
    wl3g = Wl3.T.reshape(4, 16, H)
    wr3g = Wr3.T.reshape(4, 16, H)
    w4cat = jnp.pad(
        jnp.concatenate([Wl4.T, Wr4.T], axis=1), ((0, 0), (0, 12)))
    bl1r = bl1.reshape(1, H)
    bl2r = bl2.reshape(1, H)
    bl3r = bl3.reshape(1, H)
    bl4p = jnp.pad(bl4, (0, 14)).reshape(1, 16)
    sh4 = jnp.asarray(np.eye(16, k=-2, dtype=np.float32))

    wspec = pl.BlockSpec((16, H), lambda i: (0, 0))
    wgspec = pl.BlockSpec((4, 16, H), lambda i: (0, 0, 0))
    bspec = pl.BlockSpec((1, H), lambda i: (0, 0))
    b16spec = pl.BlockSpec((1, 16), lambda i: (0, 0))
    nb16 = pl.BlockSpec((NB, 16), lambda i: (i, 0))
    hgspec = pl.BlockSpec((4, NB, 16), lambda i: (0, i, 0))
    pspec = pl.BlockSpec((2, NB, 16), lambda i: (0, i, 0))
    pgspec = pl.BlockSpec((4, 2, NB, 16), lambda i: (0, 0, i, 0))
    shspec = pl.BlockSpec((16, 16), lambda i: (0, 0))

    p1 = _seg_main(x_pad, src2d, dst2d)
    hg1, rc = pl.pallas_call(
        _dense1_body,
        grid=(NGRID,),
        in_specs=[nb16, pspec, wspec, wspec, bspec],
        out_specs=[hgspec, nb16],
        out_shape=[
            jax.ShapeDtypeStruct((4, N, 16), f32),
            jax.ShapeDtypeStruct((N, 16), f32),
        ],
    )(x_pad, p1, wl1p, wr1p, bl1r)

    def conv_mid(hg, wlg, wrg, blr, with_z):
        parts = jnp.stack([_seg_main(hg[g], src2d, dst2d) for g in range(4)])
        outs = [jax.ShapeDtypeStruct((4, N, 16), f32)]
        out_specs = [hgspec]
        if with_z:
            outs.append(jax.ShapeDtypeStruct((N, 16), f32))
            out_specs.append(nb16)
        body = _dense23z_body if with_z else _dense23n_body
        in_specs = [hgspec, pgspec, nb16, wgspec, wgspec, bspec]
        args = [hg, parts, rc, wlg, wrg, blr]
        if with_z:
            in_specs.append(shspec_w4)
            args.append(w4cat)
        return pl.pallas_call(
            body, grid=(NGRID,), in_specs=in_specs,
            out_specs=out_specs, out_shape=outs,
        )(*args)

    shspec_w4 = pl.BlockSpec((H, 16), lambda i: (0, 0))
    hg2 = conv_mid(hg1, wl2g, wr2g, bl2r, False)[0]
    hg3, z4 = conv_mid(hg2, wl3g, wr3g, bl3r, True)

    p4 = _seg_main(z4, src2d, dst2d)
    pp = pl.pallas_call(
        _dense4_body,
        grid=(NGRID,),
        in_specs=[pspec, nb16, nb16, b16spec, shspec],
        out_specs=nb16,
        out_shape=jax.ShapeDtypeStruct((N, 16), f32),
    )(p4, z4, rc, bl4p, sh4)

    pp_pad = jnp.pad(pp, ((0, EPOOL - N), (0, 0)))
    ppart = _seg_pool(pp_pad, psrc2d, pdst2d)
    out = pl.pallas_call(
        _final_body,
        grid=(1,),
        in_specs=[pl.BlockSpec((2, G, 16), lambda i: (0, 0, 0))],
        out_specs=pl.BlockSpec((G, C), lambda i: (0, 0)),
        out_shape=jax.ShapeDtypeStruct((G, C), f32),
    )(ppart)
    return out


def _dense23n_body(hgb_ref, p_ref, rc_ref, wl_ref, wr_ref, b_ref, hg_out):
    acc = jnp.broadcast_to(b_ref[...], (NB, H))
    rc = rc_ref[...]
    for g in range(4):
        mean_g = (p_ref[g, 0] + p_ref[g, 1]) * rc
        acc = acc + jnp.dot(mean_g, wl_ref[g], preferred_element_type=jnp.float32)
        acc = acc + jnp.dot(hgb_ref[g], wr_ref[g], preferred_element_type=jnp.float32)
    h = jnp.maximum(acc, 0.0)
    for g in range(4):
        hg_out[g] = h[:, g * 16:(g + 1) * 16]


def _dense23z_body(hgb_ref, p_ref, rc_ref, wl_ref, wr_ref, b_ref, w4_ref,
                   hg_out, z_out):
    acc = jnp.broadcast_to(b_ref[...], (NB, H))
    rc = rc_ref[...]
    for g in range(4):
        mean_g = (p_ref[g, 0] + p_ref[g, 1]) * rc
        acc = acc + jnp.dot(mean_g, wl_ref[g], preferred_element_type=jnp.float32)
        acc = acc + jnp.dot(hgb_ref[g], wr_ref[g], preferred_element_type=jnp.float32)
    h = jnp.maximum(acc, 0.0)
    for g in range(4):
        hg_out[g] = h[:, g * 16:(g + 1) * 16]
    z_out[...] = jnp.dot(h, w4_ref[...], preferred_element_type=jnp.float32)


# speedup vs baseline: 17.8302x; 1.0587x over previous
import functools
import math

import jax
import jax.numpy as jnp
import numpy as np
from jax import lax
from jax.experimental import pallas as pl
from jax.experimental.pallas import tpu as pltpu
from jax.experimental.pallas import tpu_sc as plsc

N = 100000
E = 3200000
F_IN = 14
H = 64
C = 2
G = 512

NC = 2
NS = 16
NW = NC * NS
LANES = 128
B = 512
JROWS = B // LANES

_EBLK = NW * B
EPAD = math.ceil(E / (2 * _EBLK)) * (2 * _EBLK)
NBLK = EPAD // _EBLK
NAPAD = 100352
EPOOL = math.ceil(N / (2 * _EBLK)) * (2 * _EBLK)
NA_POOL = 2048
NB = 1000
NGRID = N // NB


def _make_seg16(nepad, napad, linear_payload=False):
    nblk = nepad // _EBLK
    assert nblk % 4 == 0
    nsteps = nblk // 4
    za = napad // NS

    @functools.partial(
        pl.kernel,
        out_type=jax.ShapeDtypeStruct((NC, napad, 16), jnp.float32),
        mesh=plsc.VectorSubcoreMesh(core_axis_name="c", subcore_axis_name="s"),
        compiler_params=pltpu.CompilerParams(use_tc_tiling_on_sc=False),
        scratch_types=[
            pltpu.VMEM((4, JROWS, LANES), jnp.int32),
            pltpu.VMEM((4, JROWS, LANES), jnp.int32),
            pltpu.VMEM((B, 16), jnp.float32),
            pltpu.VMEM((B, 16), jnp.float32),
            pltpu.SemaphoreType.DMA,
            pltpu.SemaphoreType.DMA,
            pltpu.SemaphoreType.DMA,
            pltpu.SemaphoreType.DMA,
            pltpu.SemaphoreType.DMA,
            pltpu.SemaphoreType.DMA,
            pltpu.SemaphoreType.DMA,
            pltpu.SemaphoreType.DMA,
            pltpu.VMEM_SHARED((napad, 16), jnp.float32),
        ],
    )
    def seg(tab, srcr, dstr, out, sv, dv, rows0, rows1,
            semi0, semi1, semi2, semi3, semg0, semg1, sems0, sems1, acc):
        c = lax.axis_index("c")
        s = lax.axis_index("s")
        w = s * NC + c
        base = w * nblk

        rows = (rows0, rows1)
        semi = (semi0, semi1, semi2, semi3)
        semg = (semg0, semg1)
        sems = (sems0, sems1)

        def idx_issue(blk_i, q):
            r0 = (base + blk_i) * JROWS
            pltpu.async_copy(srcr.at[pl.ds(r0, JROWS)], sv.at[q], semi[q])
            pltpu.async_copy(dstr.at[pl.ds(r0, JROWS)], dv.at[q], semi[q])

        def idx_wait(q):
            pltpu.make_async_copy(
                srcr.at[pl.ds(0, JROWS)], sv.at[q], semi[q]).wait()
            pltpu.make_async_copy(
                dstr.at[pl.ds(0, JROWS)], dv.at[q], semi[q]).wait()

        def gather_issue(blk_i, p, q):
            if linear_payload:
                r0 = (base + blk_i) * B
                pltpu.async_copy(tab.at[pl.ds(r0, B)], rows[p], semg[p])
            else:
                for j in range(JROWS):
                    pltpu.async_copy(
                        tab.at[sv.at[q, j]],
                        rows[p].at[pl.ds(j * LANES, LANES)],
                        semg[p],
                    )

        def gather_wait(p, q):
            if linear_payload:
                pltpu.make_async_copy(
                    tab.at[pl.ds(0, B)], rows[p], semg[p]).wait()
            else:
                for j in range(JROWS):
                    pltpu.make_async_copy(
                        tab.at[sv.at[q, j]],
                        rows[p].at[pl.ds(j * LANES, LANES)],
                        semg[p],
                    ).wait()

        def scatter_issue(p, q):
            for j in range(JROWS):
                pltpu.async_copy(
                    rows[p].at[pl.ds(j * LANES, LANES)],
                    acc.at[dv.at[q, j]],
                    sems[p],
                    add=True,
                )

        def scatter_wait(p, q):
            for j in range(JROWS):
                pltpu.make_async_copy(
                    rows[p].at[pl.ds(j * LANES, LANES)],
                    acc.at[dv.at[q, j]],
                    sems[p],
                ).wait()

        def zb(m, carry):
            rows0[m, :] = jnp.zeros((16,), jnp.float32)
            return carry

        lax.fori_loop(0, LANES, zb, 0)

        def zc(k, carry):
            pltpu.async_copy(rows0.at[pl.ds(0, LANES)],
                             acc.at[pl.ds(s * za + k * LANES, LANES)], semg0)
            return carry

        lax.fori_loop(0, za // LANES, zc, 0)

        def zw(k, carry):
            pltpu.make_async_copy(
                rows0.at[pl.ds(0, LANES)],
                acc.at[pl.ds(s * za, LANES)], semg0).wait()
            return carry

        lax.fori_loop(0, za // LANES, zw, 0)
        plsc.subcore_barrier()

        def clamp(i):
            return jnp.minimum(i, nblk - 1)

        idx_issue(0, 0)
        idx_issue(1, 1)
        idx_wait(0)
        gather_issue(0, 0, 0)
        idx_issue(2, 2)
        idx_wait(1)
        gather_issue(1, 1, 1)
        idx_issue(3, 3)
        gather_wait(0, 0)
        scatter_issue(0, 0)
        idx_wait(2)
        scatter_wait(0, 0)
        gather_issue(2, 0, 2)
        idx_issue(clamp(4), 0)
        gather_wait(1, 1)
        scatter_issue(1, 1)
        idx_wait(3)
        scatter_wait(1, 1)
        gather_issue(3, 1, 3)
        idx_issue(clamp(5), 1)
        gather_wait(0, 2)
        scatter_issue(0, 2)

        def step(t, carry):
            b0 = 4 * t
            idx_wait(0)
            scatter_wait(0, 2)
            gather_issue(b0, 0, 0)
            idx_issue(clamp(b0 + 2), 2)
            gather_wait(1, 3)
            scatter_issue(1, 3)
            idx_wait(1)
            scatter_wait(1, 3)
            gather_issue(b0 + 1, 1, 1)
            idx_issue(clamp(b0 + 3), 3)
            gather_wait(0, 0)
            scatter_issue(0, 0)
            idx_wait(2)
            scatter_wait(0, 0)
            gather_issue(b0 + 2, 0, 2)
            idx_issue(clamp(b0 + 4), 0)
            gather_wait(1, 1)
            scatter_issue(1, 1)
            idx_wait(3)
            scatter_wait(1, 1)
            gather_issue(b0 + 3, 1, 3)
            idx_issue(clamp(b0 + 5), 1)
            gather_wait(0, 2)
            scatter_issue(0, 2)
            return carry

        lax.fori_loop(1, nsteps, step, 0)

        gather_wait(1, 3)
        scatter_issue(1, 3)
        scatter_wait(0, 2)
        scatter_wait(1, 3)
        idx_wait(0)
        idx_wait(1)

        plsc.subcore_barrier()
        pltpu.sync_copy(acc.at[pl.ds(s * za, za)], out.at[c, pl.ds(s * za, za)])

    return seg


_seg_main = _make_seg16(EPAD, NAPAD)
_seg_pool = _make_seg16(EPOOL, NA_POOL, linear_payload=True)


def _dense1_body(x_ref, p_ref, wl_ref, wr_ref, b_ref, hg_ref, rc_ref):
    agg = p_ref[0] + p_ref[1]
    cnt = agg[:, 14]
    rc = 1.0 / jnp.maximum(cnt, 1.0)
    mean = agg * rc[:, None]
    h = jnp.dot(mean, wl_ref[...], preferred_element_type=jnp.float32)
    h = h + jnp.dot(x_ref[...], wr_ref[...], preferred_element_type=jnp.float32)
    h = jnp.maximum(h + b_ref[...], 0.0)
    for g in range(4):
        hg_ref[g] = h[:, g * 16:(g + 1) * 16]
    rc_ref[...] = jnp.broadcast_to(rc[:, None], (NB, 16))


def _dense4_body(p_ref, z_ref, rc_ref, b_ref, sh_ref, out_ref):
    s = p_ref[0] + p_ref[1]
    t = s * rc_ref[...]
    col = lax.broadcasted_iota(jnp.int32, (NB, 16), 1)
    m01 = (col < 2).astype(jnp.float32)
    zsh = jnp.dot(z_ref[...], sh_ref[...], preferred_element_type=jnp.float32)
    out_ref[...] = t * m01 + b_ref[...] + zsh + (col == 2).astype(jnp.float32)


def _final_body(p_ref, out_ref):
    s = p_ref[0] + p_ref[1]
    cnt = s[:, 2]
    rcg = 1.0 / jnp.maximum(cnt, 1.0)
    col = lax.broadcasted_iota(jnp.int32, (G, 16), 1)
    pooled = jnp.where(col < 2, s * rcg[:, None], -1e30)
    m = jnp.max(pooled, axis=1, keepdims=True)
    lse = m + jnp.log(jnp.sum(jnp.exp(pooled - m), axis=1, keepdims=True))
    out_ref[...] = (pooled - lse)[:, 0:2]


def _pad_edges(src, dst, nepad, trash_base):
    npad = nepad - src.shape[0]
    fill = jnp.arange(npad, dtype=jnp.int32) % 8
    srcp = jnp.concatenate([src, fill])
    dstp = jnp.concatenate([dst, trash_base + fill])
    return srcp.reshape(-1, LANES), dstp.reshape(-1, LANES)


def kernel(x, edge_index, batch, Wl1, bl1, Wr1, Wl2, bl2, Wr2, Wl3, bl3, Wr3,
           Wl4, bl4, Wr4):
    f32 = jnp.float32
    src = edge_index[0]
    dst = edge_index[1]
    src2d, dst2d = _pad_edges(src, dst, EPAD, N)
    psrc2d, pdst2d = _pad_edges(
        jnp.arange(N, dtype=jnp.int32), batch, EPOOL, G)

    ones_col = jnp.ones((N, 1), f32)
    zeros_col = jnp.zeros((N, 1), f32)
    x_pad = jnp.concatenate([x, ones_col, zeros_col], axis=1)

    wl1p = jnp.pad(Wl1.T, ((0, 2), (0, 0)))
    wr1p = jnp.pad(Wr1.T, ((0, 2), (0, 0)))
    wl2g = Wl2.T.reshape(4, 16, H)
    wr2g = Wr2.T.reshape(4, 16, H)
    wl3g = Wl3.T.reshape(4, 16, H)
    wr3g = Wr3.T.reshape(4, 16, H)
    w4cat = jnp.pad(
        jnp.concatenate([Wl4.T, Wr4.T], axis=1), ((0, 0), (0, 12)))
    bl1r = bl1.reshape(1, H)
    bl2r = bl2.reshape(1, H)
    bl3r = bl3.reshape(1, H)
    bl4p = jnp.pad(bl4, (0, 14)).reshape(1, 16)
    sh4 = jnp.asarray(np.eye(16, k=-2, dtype=np.float32))

    wspec = pl.BlockSpec((16, H), lambda i: (0, 0))
    wgspec = pl.BlockSpec((4, 16, H), lambda i: (0, 0, 0))
    bspec = pl.BlockSpec((1, H), lambda i: (0, 0))
    b16spec = pl.BlockSpec((1, 16), lambda i: (0, 0))
    nb16 = pl.BlockSpec((NB, 16), lambda i: (i, 0))
    hgspec = pl.BlockSpec((4, NB, 16), lambda i: (0, i, 0))
    pspec = pl.BlockSpec((2, NB, 16), lambda i: (0, i, 0))
    pgspec = pl.BlockSpec((4, 2, NB, 16), lambda i: (0, 0, i, 0))
    shspec = pl.BlockSpec((16, 16), lambda i: (0, 0))

    p1 = _seg_main(x_pad, src2d, dst2d)
    hg1, rc = pl.pallas_call(
        _dense1_body,
        grid=(NGRID,),
        in_specs=[nb16, pspec, wspec, wspec, bspec],
        out_specs=[hgspec, nb16],
        out_shape=[
            jax.ShapeDtypeStruct((4, N, 16), f32),
            jax.ShapeDtypeStruct((N, 16), f32),
        ],
    )(x_pad, p1, wl1p, wr1p, bl1r)

    def conv_mid(hg, wlg, wrg, blr, with_z):
        parts = jnp.stack([_seg_main(hg[g], src2d, dst2d) for g in range(4)])
        outs = [jax.ShapeDtypeStruct((4, N, 16), f32)]
        out_specs = [hgspec]
        if with_z:
            outs.append(jax.ShapeDtypeStruct((N, 16), f32))
            out_specs.append(nb16)
        body = _dense23z_body if with_z else _dense23n_body
        in_specs = [hgspec, pgspec, nb16, wgspec, wgspec, bspec]
        args = [hg, parts, rc, wlg, wrg, blr]
        if with_z:
            in_specs.append(shspec_w4)
            args.append(w4cat)
        return pl.pallas_call(
            body, grid=(NGRID,), in_specs=in_specs,
            out_specs=out_specs, out_shape=outs,
        )(*args)

    shspec_w4 = pl.BlockSpec((H, 16), lambda i: (0, 0))
    hg2 = conv_mid(hg1, wl2g, wr2g, bl2r, False)[0]
    hg3, z4 = conv_mid(hg2, wl3g, wr3g, bl3r, True)

    p4 = _seg_main(z4, src2d, dst2d)
    pp = pl.pallas_call(
        _dense4_body,
        grid=(NGRID,),
        in_specs=[pspec, nb16, nb16, b16spec, shspec],
        out_specs=nb16,
        out_shape=jax.ShapeDtypeStruct((N, 16), f32),
    )(p4, z4, rc, bl4p, sh4)

    pp_pad = jnp.pad(pp, ((0, EPOOL - N), (0, 0)))
    ppart = _seg_pool(pp_pad, psrc2d, pdst2d)
    out = pl.pallas_call(
        _final_body,
        grid=(1,),
        in_specs=[pl.BlockSpec((2, G, 16), lambda i: (0, 0, 0))],
        out_specs=pl.BlockSpec((G, C), lambda i: (0, 0)),
        out_shape=jax.ShapeDtypeStruct((G, C), f32),
    )(ppart)
    return out


def _dense23n_body(hgb_ref, p_ref, rc_ref, wl_ref, wr_ref, b_ref, hg_out):
    acc = jnp.broadcast_to(b_ref[...], (NB, H))
    rc = rc_ref[...]
    for g in range(4):
        mean_g = (p_ref[g, 0] + p_ref[g, 1]) * rc
        acc = acc + jnp.dot(mean_g, wl_ref[g], preferred_element_type=jnp.float32)
        acc = acc + jnp.dot(hgb_ref[g], wr_ref[g], preferred_element_type=jnp.float32)
    h = jnp.maximum(acc, 0.0)
    for g in range(4):
        hg_out[g] = h[:, g * 16:(g + 1) * 16]


def _dense23z_body(hgb_ref, p_ref, rc_ref, wl_ref, wr_ref, b_ref, w4_ref,
                   hg_out, z_out):
    acc = jnp.broadcast_to(b_ref[...], (NB, H))
    rc = rc_ref[...]
    for g in range(4):
        mean_g = (p_ref[g, 0] + p_ref[g, 1]) * rc
        acc = acc + jnp.dot(mean_g, wl_ref[g], preferred_element_type=jnp.float32)
        acc = acc + jnp.dot(hgb_ref[g], wr_ref[g], preferred_element_type=jnp.float32)
    h = jnp.maximum(acc, 0.0)
    for g in range(4):
        hg_out[g] = h[:, g * 16:(g + 1) * 16]
    z_out[...] = jnp.dot(h, w4_ref[...], preferred_element_type=jnp.float32)


# speedup vs baseline: 19.1989x; 1.0768x over previous
import functools
import math

import jax
import jax.numpy as jnp
import numpy as np
from jax import lax
from jax.experimental import pallas as pl
from jax.experimental.pallas import tpu as pltpu
from jax.experimental.pallas import tpu_sc as plsc

N = 100000
E = 3200000
F_IN = 14
H = 64
C = 2
G = 512

NC = 2
NS = 16
NW = NC * NS
LANES = 128
B = 512
JROWS = B // LANES

_EBLK = NW * B
EPAD = math.ceil(E / (2 * _EBLK)) * (2 * _EBLK)
NBLK = EPAD // _EBLK
NAPAD = 100352
EPOOL = math.ceil(N / (2 * _EBLK)) * (2 * _EBLK)
NA_POOL = 2048
NB = 1000
NGRID = N // NB


def _make_seg16(nepad, napad, ngroups=1, linear_payload=False):
    nblk = nepad // _EBLK
    assert nblk % 4 == 0
    nsteps = nblk // 4
    za = napad // NS

    @functools.partial(
        pl.kernel,
        out_type=jax.ShapeDtypeStruct((ngroups, NC, napad, 16), jnp.float32),
        mesh=plsc.VectorSubcoreMesh(core_axis_name="c", subcore_axis_name="s"),
        compiler_params=pltpu.CompilerParams(use_tc_tiling_on_sc=False),
        scratch_types=[
            pltpu.VMEM((4, 2 * JROWS, LANES), jnp.int32),
            pltpu.VMEM((B, 16), jnp.float32),
            pltpu.VMEM((B, 16), jnp.float32),
            pltpu.SemaphoreType.DMA,
            pltpu.SemaphoreType.DMA,
            pltpu.SemaphoreType.DMA,
            pltpu.SemaphoreType.DMA,
            pltpu.SemaphoreType.DMA,
            pltpu.SemaphoreType.DMA,
            pltpu.SemaphoreType.DMA,
            pltpu.SemaphoreType.DMA,
            pltpu.VMEM_SHARED((napad, 16), jnp.float32),
        ],
    )
    def seg(tab, edg, out, ev, rows0, rows1,
            semi0, semi1, semi2, semi3, semg0, semg1, sems0, sems1, acc):
        c = lax.axis_index("c")
        s = lax.axis_index("s")
        w = s * NC + c
        base = w * nblk

        rows = (rows0, rows1)
        semi = (semi0, semi1, semi2, semi3)
        semg = (semg0, semg1)
        sems = (sems0, sems1)

        def idx_issue(blk_i, q):
            pltpu.async_copy(edg.at[base + blk_i], ev.at[q], semi[q])

        def idx_wait(q):
            pltpu.make_async_copy(edg.at[0], ev.at[q], semi[q]).wait()

        def body_for_group(g):
            tabg = tab.at[g]

            def gather_issue(blk_i, p, q):
                if linear_payload:
                    r0 = (base + blk_i) * B
                    pltpu.async_copy(tabg.at[pl.ds(r0, B)], rows[p], semg[p])
                else:
                    for j in range(JROWS):
                        pltpu.async_copy(
                            tabg.at[ev.at[q, j]],
                            rows[p].at[pl.ds(j * LANES, LANES)],
                            semg[p],
                        )

            def gather_wait(p, q):
                if linear_payload:
                    pltpu.make_async_copy(
                        tabg.at[pl.ds(0, B)], rows[p], semg[p]).wait()
                else:
                    for j in range(JROWS):
                        pltpu.make_async_copy(
                            tabg.at[ev.at[q, j]],
                            rows[p].at[pl.ds(j * LANES, LANES)],
                            semg[p],
                        ).wait()

            def scatter_issue(p, q):
                for j in range(JROWS):
                    pltpu.async_copy(
                        rows[p].at[pl.ds(j * LANES, LANES)],
                        acc.at[ev.at[q, JROWS + j]],
                        sems[p],
                        add=True,
                    )

            def scatter_wait(p, q):
                for j in range(JROWS):
                    pltpu.make_async_copy(
                        rows[p].at[pl.ds(j * LANES, LANES)],
                        acc.at[ev.at[q, JROWS + j]],
                        sems[p],
                    ).wait()

            def zb(m, carry):
                rows1[m, :] = jnp.zeros((16,), jnp.float32)
                return carry

            lax.fori_loop(0, LANES, zb, 0)

            def zc(k, carry):
                pltpu.async_copy(rows1.at[pl.ds(0, LANES)],
                                 acc.at[pl.ds(s * za + k * LANES, LANES)],
                                 semg0)
                return carry

            lax.fori_loop(0, za // LANES, zc, 0)

            def zw(k, carry):
                pltpu.make_async_copy(
                    rows1.at[pl.ds(0, LANES)],
                    acc.at[pl.ds(s * za, LANES)], semg0).wait()
                return carry

            lax.fori_loop(0, za // LANES, zw, 0)
            plsc.subcore_barrier()

            def clamp(i):
                return jnp.minimum(i, nblk - 1)

            idx_issue(0, 0)
            idx_issue(1, 1)
            idx_wait(0)
            gather_issue(0, 0, 0)
            idx_issue(2, 2)
            idx_wait(1)
            gather_issue(1, 1, 1)
            idx_issue(3, 3)
            gather_wait(0, 0)
            scatter_issue(0, 0)
            idx_wait(2)
            scatter_wait(0, 0)
            gather_issue(2, 0, 2)
            idx_issue(clamp(4), 0)
            gather_wait(1, 1)
            scatter_issue(1, 1)
            idx_wait(3)
            scatter_wait(1, 1)
            gather_issue(3, 1, 3)
            idx_issue(clamp(5), 1)
            gather_wait(0, 2)
            scatter_issue(0, 2)

            def step(t, carry):
                b0 = 4 * t
                idx_wait(0)
                scatter_wait(0, 2)
                gather_issue(b0, 0, 0)
                idx_issue(clamp(b0 + 2), 2)
                gather_wait(1, 3)
                scatter_issue(1, 3)
                idx_wait(1)
                scatter_wait(1, 3)
                gather_issue(b0 + 1, 1, 1)
                idx_issue(clamp(b0 + 3), 3)
                gather_wait(0, 0)
                scatter_issue(0, 0)
                idx_wait(2)
                scatter_wait(0, 0)
                gather_issue(b0 + 2, 0, 2)
                idx_issue(clamp(b0 + 4), 0)
                gather_wait(1, 1)
                scatter_issue(1, 1)
                idx_wait(3)
                scatter_wait(1, 1)
                gather_issue(b0 + 3, 1, 3)
                idx_issue(clamp(b0 + 5), 1)
                gather_wait(0, 2)
                scatter_issue(0, 2)
                return carry

            lax.fori_loop(1, nsteps, step, 0)

            gather_wait(1, 3)
            scatter_issue(1, 3)
            scatter_wait(0, 2)
            scatter_wait(1, 3)
            idx_wait(0)
            idx_wait(1)

            plsc.subcore_barrier()
            pltpu.sync_copy(acc.at[pl.ds(s * za, za)],
                            out.at[g, c, pl.ds(s * za, za)])

        for g in range(ngroups):
            body_for_group(g)

    return seg


_seg_one = _make_seg16(EPAD, NAPAD, 1)
_seg_four = _make_seg16(EPAD, NAPAD, 4)
_seg_pool = _make_seg16(EPOOL, NA_POOL, 1, linear_payload=True)


def _dense1_body(x_ref, p_ref, wl_ref, wr_ref, b_ref, hg_ref, rc_ref):
    agg = p_ref[0] + p_ref[1]
    cnt = agg[:, 14]
    rc = 1.0 / jnp.maximum(cnt, 1.0)
    mean = agg * rc[:, None]
    h = jnp.dot(mean, wl_ref[...], preferred_element_type=jnp.float32)
    h = h + jnp.dot(x_ref[...], wr_ref[...], preferred_element_type=jnp.float32)
    h = jnp.maximum(h + b_ref[...], 0.0)
    for g in range(4):
        hg_ref[g] = h[:, g * 16:(g + 1) * 16]
    rc_ref[...] = jnp.broadcast_to(rc[:, None], (NB, 16))


def _dense4_body(p_ref, z_ref, rc_ref, b_ref, sh_ref, out_ref):
    s = p_ref[0] + p_ref[1]
    t = s * rc_ref[...]
    col = lax.broadcasted_iota(jnp.int32, (NB, 16), 1)
    m01 = (col < 2).astype(jnp.float32)
    zsh = jnp.dot(z_ref[...], sh_ref[...], preferred_element_type=jnp.float32)
    out_ref[...] = t * m01 + b_ref[...] + zsh + (col == 2).astype(jnp.float32)


def _final_body(p_ref, out_ref):
    s = p_ref[0] + p_ref[1]
    cnt = s[:, 2]
    rcg = 1.0 / jnp.maximum(cnt, 1.0)
    col = lax.broadcasted_iota(jnp.int32, (G, 16), 1)
    pooled = jnp.where(col < 2, s * rcg[:, None], -1e30)
    m = jnp.max(pooled, axis=1, keepdims=True)
    lse = m + jnp.log(jnp.sum(jnp.exp(pooled - m), axis=1, keepdims=True))
    out_ref[...] = (pooled - lse)[:, 0:2]


def _pad_edges(src, dst, nepad, trash_base):
    npad = nepad - src.shape[0]
    fill = jnp.arange(npad, dtype=jnp.int32) % 8
    srcp = jnp.concatenate([src, fill]).reshape(-1, JROWS, LANES)
    dstp = jnp.concatenate([dst, trash_base + fill]).reshape(-1, JROWS, LANES)
    return jnp.concatenate([srcp, dstp], axis=1)


def kernel(x, edge_index, batch, Wl1, bl1, Wr1, Wl2, bl2, Wr2, Wl3, bl3, Wr3,
           Wl4, bl4, Wr4):
    f32 = jnp.float32
    src = edge_index[0]
    dst = edge_index[1]
    e2 = _pad_edges(src, dst, EPAD, N)
    pe2 = _pad_edges(jnp.arange(N, dtype=jnp.int32), batch, EPOOL, G)

    ones_col = jnp.ones((N, 1), f32)
    zeros_col = jnp.zeros((N, 1), f32)
    x_pad = jnp.concatenate([x, ones_col, zeros_col], axis=1)

    wl1p = jnp.pad(Wl1.T, ((0, 2), (0, 0)))
    wr1p = jnp.pad(Wr1.T, ((0, 2), (0, 0)))
    wl2g = Wl2.T.reshape(4, 16, H)
    wr2g = Wr2.T.reshape(4, 16, H)
    wl3g = Wl3.T.reshape(4, 16, H)
    wr3g = Wr3.T.reshape(4, 16, H)
    w4cat = jnp.pad(
        jnp.concatenate([Wl4.T, Wr4.T], axis=1), ((0, 0), (0, 12)))
    bl1r = bl1.reshape(1, H)
    bl2r = bl2.reshape(1, H)
    bl3r = bl3.reshape(1, H)
    bl4p = jnp.pad(bl4, (0, 14)).reshape(1, 16)
    sh4 = jnp.asarray(np.eye(16, k=-2, dtype=np.float32))

    wspec = pl.BlockSpec((16, H), lambda i: (0, 0))
    wgspec = pl.BlockSpec((4, 16, H), lambda i: (0, 0, 0))
    bspec = pl.BlockSpec((1, H), lambda i: (0, 0))
    b16spec = pl.BlockSpec((1, 16), lambda i: (0, 0))
    nb16 = pl.BlockSpec((NB, 16), lambda i: (i, 0))
    hgspec = pl.BlockSpec((4, NB, 16), lambda i: (0, i, 0))
    pspec = pl.BlockSpec((2, NB, 16), lambda i: (0, i, 0))
    pgspec = pl.BlockSpec((4, 2, NB, 16), lambda i: (0, 0, i, 0))
    shspec = pl.BlockSpec((16, 16), lambda i: (0, 0))

    p1 = jnp.squeeze(_seg_one(x_pad[None], e2), axis=0)
    hg1, rc = pl.pallas_call(
        _dense1_body,
        grid=(NGRID,),
        in_specs=[nb16, pspec, wspec, wspec, bspec],
        out_specs=[hgspec, nb16],
        out_shape=[
            jax.ShapeDtypeStruct((4, N, 16), f32),
            jax.ShapeDtypeStruct((N, 16), f32),
        ],
    )(x_pad, p1, wl1p, wr1p, bl1r)

    def conv_mid(hg, wlg, wrg, blr, with_z):
        parts = _seg_four(hg, e2)
        outs = [jax.ShapeDtypeStruct((4, N, 16), f32)]
        out_specs = [hgspec]
        if with_z:
            outs.append(jax.ShapeDtypeStruct((N, 16), f32))
            out_specs.append(nb16)
        body = _dense23z_body if with_z else _dense23n_body
        in_specs = [hgspec, pgspec, nb16, wgspec, wgspec, bspec]
        args = [hg, parts, rc, wlg, wrg, blr]
        if with_z:
            in_specs.append(shspec_w4)
            args.append(w4cat)
        return pl.pallas_call(
            body, grid=(NGRID,), in_specs=in_specs,
            out_specs=out_specs, out_shape=outs,
        )(*args)

    shspec_w4 = pl.BlockSpec((H, 16), lambda i: (0, 0))
    hg2 = conv_mid(hg1, wl2g, wr2g, bl2r, False)[0]
    hg3, z4 = conv_mid(hg2, wl3g, wr3g, bl3r, True)

    p4 = jnp.squeeze(_seg_one(z4[None], e2), axis=0)
    pp = pl.pallas_call(
        _dense4_body,
        grid=(NGRID,),
        in_specs=[pspec, nb16, nb16, b16spec, shspec],
        out_specs=nb16,
        out_shape=jax.ShapeDtypeStruct((N, 16), f32),
    )(p4, z4, rc, bl4p, sh4)

    pp_pad = jnp.pad(pp, ((0, EPOOL - N), (0, 0)))
    ppart = jnp.squeeze(_seg_pool(pp_pad[None], pe2), axis=0)
    out = pl.pallas_call(
        _final_body,
        grid=(1,),
        in_specs=[pl.BlockSpec((2, G, 16), lambda i: (0, 0, 0))],
        out_specs=pl.BlockSpec((G, C), lambda i: (0, 0)),
        out_shape=jax.ShapeDtypeStruct((G, C), f32),
    )(ppart)
    return out


def _dense23n_body(hgb_ref, p_ref, rc_ref, wl_ref, wr_ref, b_ref, hg_out):
    acc = jnp.broadcast_to(b_ref[...], (NB, H))
    rc = rc_ref[...]
    for g in range(4):
        mean_g = (p_ref[g, 0] + p_ref[g, 1]) * rc
        acc = acc + jnp.dot(mean_g, wl_ref[g], preferred_element_type=jnp.float32)
        acc = acc + jnp.dot(hgb_ref[g], wr_ref[g], preferred_element_type=jnp.float32)
    h = jnp.maximum(acc, 0.0)
    for g in range(4):
        hg_out[g] = h[:, g * 16:(g + 1) * 16]


def _dense23z_body(hgb_ref, p_ref, rc_ref, wl_ref, wr_ref, b_ref, w4_ref,
                   hg_out, z_out):
    acc = jnp.broadcast_to(b_ref[...], (NB, H))
    rc = rc_ref[...]
    for g in range(4):
        mean_g = (p_ref[g, 0] + p_ref[g, 1]) * rc
        acc = acc + jnp.dot(mean_g, wl_ref[g], preferred_element_type=jnp.float32)
        acc = acc + jnp.dot(hgb_ref[g], wr_ref[g], preferred_element_type=jnp.float32)
    h = jnp.maximum(acc, 0.0)
    for g in range(4):
        hg_out[g] = h[:, g * 16:(g + 1) * 16]
    z_out[...] = jnp.dot(h, w4_ref[...], preferred_element_type=jnp.float32)


# speedup vs baseline: 19.7854x; 1.0306x over previous
import functools
import math

import jax
import jax.numpy as jnp
import numpy as np
from jax import lax
from jax.experimental import pallas as pl
from jax.experimental.pallas import tpu as pltpu
from jax.experimental.pallas import tpu_sc as plsc

N = 100000
E = 3200000
F_IN = 14
H = 64
C = 2
G = 512

NC = 2
NS = 16
NW = NC * NS
LANES = 128
B = 512
JROWS = B // LANES

_EBLK = NW * B
EPAD = math.ceil(E / (2 * _EBLK)) * (2 * _EBLK)
NBLK = EPAD // _EBLK
NAPAD = 100352
EPOOL = math.ceil(N / (2 * _EBLK)) * (2 * _EBLK)
NA_POOL = 2048
NB = 2000
NGRID = N // NB


def _make_seg16(nepad, napad, ngroups=1, linear_payload=False):
    nblk = nepad // _EBLK
    assert nblk % 4 == 0
    nsteps = nblk // 4
    za = napad // NS

    @functools.partial(
        pl.kernel,
        out_type=jax.ShapeDtypeStruct((ngroups, NC, napad, 16), jnp.float32),
        mesh=plsc.VectorSubcoreMesh(core_axis_name="c", subcore_axis_name="s"),
        compiler_params=pltpu.CompilerParams(use_tc_tiling_on_sc=False),
        scratch_types=[
            pltpu.VMEM((4, 2 * JROWS, LANES), jnp.int32),
            pltpu.VMEM((B, 16), jnp.float32),
            pltpu.VMEM((B, 16), jnp.float32),
            pltpu.SemaphoreType.DMA,
            pltpu.SemaphoreType.DMA,
            pltpu.SemaphoreType.DMA,
            pltpu.SemaphoreType.DMA,
            pltpu.SemaphoreType.DMA,
            pltpu.SemaphoreType.DMA,
            pltpu.SemaphoreType.DMA,
            pltpu.SemaphoreType.DMA,
            pltpu.VMEM_SHARED((napad, 16), jnp.float32),
        ],
    )
    def seg(tab, edg, out, ev, rows0, rows1,
            semi0, semi1, semi2, semi3, semg0, semg1, sems0, sems1, acc):
        c = lax.axis_index("c")
        s = lax.axis_index("s")
        w = s * NC + c
        base = w * nblk

        rows = (rows0, rows1)
        semi = (semi0, semi1, semi2, semi3)
        semg = (semg0, semg1)
        sems = (sems0, sems1)

        def idx_issue(blk_i, q):
            pltpu.async_copy(edg.at[base + blk_i], ev.at[q], semi[q])

        def idx_wait(q):
            pltpu.make_async_copy(edg.at[0], ev.at[q], semi[q]).wait()

        def body_for_group(g):
            tabg = tab.at[g]

            def gather_issue(blk_i, p, q):
                if linear_payload:
                    r0 = (base + blk_i) * B
                    pltpu.async_copy(tabg.at[pl.ds(r0, B)], rows[p], semg[p])
                else:
                    for j in range(JROWS):
                        pltpu.async_copy(
                            tabg.at[ev.at[q, j]],
                            rows[p].at[pl.ds(j * LANES, LANES)],
                            semg[p],
                        )

            def gather_wait(p, q):
                if linear_payload:
                    pltpu.make_async_copy(
                        tabg.at[pl.ds(0, B)], rows[p], semg[p]).wait()
                else:
                    for j in range(JROWS):
                        pltpu.make_async_copy(
                            tabg.at[ev.at[q, j]],
                            rows[p].at[pl.ds(j * LANES, LANES)],
                            semg[p],
                        ).wait()

            def scatter_issue(p, q):
                for j in range(JROWS):
                    pltpu.async_copy(
                        rows[p].at[pl.ds(j * LANES, LANES)],
                        acc.at[ev.at[q, JROWS + j]],
                        sems[p],
                        add=True,
                    )

            def scatter_wait(p, q):
                for j in range(JROWS):
                    pltpu.make_async_copy(
                        rows[p].at[pl.ds(j * LANES, LANES)],
                        acc.at[ev.at[q, JROWS + j]],
                        sems[p],
                    ).wait()

            def zb(m, carry):
                rows1[m, :] = jnp.zeros((16,), jnp.float32)
                return carry

            lax.fori_loop(0, LANES, zb, 0)

            def zc(k, carry):
                pltpu.async_copy(rows1.at[pl.ds(0, LANES)],
                                 acc.at[pl.ds(s * za + k * LANES, LANES)],
                                 semg0)
                return carry

            lax.fori_loop(0, za // LANES, zc, 0)

            def zw(k, carry):
                pltpu.make_async_copy(
                    rows1.at[pl.ds(0, LANES)],
                    acc.at[pl.ds(s * za, LANES)], semg0).wait()
                return carry

            lax.fori_loop(0, za // LANES, zw, 0)
            plsc.subcore_barrier()

            def clamp(i):
                return jnp.minimum(i, nblk - 1)

            idx_issue(0, 0)
            idx_issue(1, 1)
            idx_wait(0)
            gather_issue(0, 0, 0)
            idx_issue(2, 2)
            idx_wait(1)
            gather_issue(1, 1, 1)
            idx_issue(3, 3)
            gather_wait(0, 0)
            scatter_issue(0, 0)
            idx_wait(2)
            scatter_wait(0, 0)
            gather_issue(2, 0, 2)
            idx_issue(clamp(4), 0)
            gather_wait(1, 1)
            scatter_issue(1, 1)
            idx_wait(3)
            scatter_wait(1, 1)
            gather_issue(3, 1, 3)
            idx_issue(clamp(5), 1)
            gather_wait(0, 2)
            scatter_issue(0, 2)

            def step(t, carry):
                b0 = 4 * t
                idx_wait(0)
                scatter_wait(0, 2)
                gather_issue(b0, 0, 0)
                idx_issue(clamp(b0 + 2), 2)
                gather_wait(1, 3)
                scatter_issue(1, 3)
                idx_wait(1)
                scatter_wait(1, 3)
                gather_issue(b0 + 1, 1, 1)
                idx_issue(clamp(b0 + 3), 3)
                gather_wait(0, 0)
                scatter_issue(0, 0)
                idx_wait(2)
                scatter_wait(0, 0)
                gather_issue(b0 + 2, 0, 2)
                idx_issue(clamp(b0 + 4), 0)
                gather_wait(1, 1)
                scatter_issue(1, 1)
                idx_wait(3)
                scatter_wait(1, 1)
                gather_issue(b0 + 3, 1, 3)
                idx_issue(clamp(b0 + 5), 1)
                gather_wait(0, 2)
                scatter_issue(0, 2)
                return carry

            lax.fori_loop(1, nsteps, step, 0)

            gather_wait(1, 3)
            scatter_issue(1, 3)
            scatter_wait(0, 2)
            scatter_wait(1, 3)
            idx_wait(0)
            idx_wait(1)

            plsc.subcore_barrier()
            pltpu.sync_copy(acc.at[pl.ds(s * za, za)],
                            out.at[g, c, pl.ds(s * za, za)])

        for g in range(ngroups):
            body_for_group(g)

    return seg


_seg_one = _make_seg16(EPAD, NAPAD, 1)
_seg_four = _make_seg16(EPAD, NAPAD, 4)
_seg_pool = _make_seg16(EPOOL, NA_POOL, 1, linear_payload=True)


def _dense1_body(x_ref, p_ref, wl_ref, wr_ref, b_ref, hg_ref, rc_ref):
    agg = p_ref[0, 0] + p_ref[0, 1]
    cnt = agg[:, 14]
    rc = 1.0 / jnp.maximum(cnt, 1.0)
    mean = agg * rc[:, None]
    h = jnp.dot(mean, wl_ref[...], preferred_element_type=jnp.float32)
    h = h + jnp.dot(x_ref[...], wr_ref[...], preferred_element_type=jnp.float32)
    h = jnp.maximum(h + b_ref[...], 0.0)
    for g in range(4):
        hg_ref[g] = h[:, g * 16:(g + 1) * 16]
    rc_ref[...] = jnp.broadcast_to(rc[:, None], (NB, 16))


def _dense4_body(p_ref, z_ref, rc_ref, b_ref, sh_ref, out_ref):
    s = p_ref[0, 0] + p_ref[0, 1]
    t = s * rc_ref[...]
    col = lax.broadcasted_iota(jnp.int32, (NB, 16), 1)
    m01 = (col < 2).astype(jnp.float32)
    zsh = jnp.dot(z_ref[...], sh_ref[...], preferred_element_type=jnp.float32)
    out_ref[...] = t * m01 + b_ref[...] + zsh + (col == 2).astype(jnp.float32)


def _final_body(p_ref, out_ref):
    s = p_ref[0, 0] + p_ref[0, 1]
    cnt = s[:, 2]
    rcg = 1.0 / jnp.maximum(cnt, 1.0)
    col = lax.broadcasted_iota(jnp.int32, (G, 16), 1)
    pooled = jnp.where(col < 2, s * rcg[:, None], -1e30)
    m = jnp.max(pooled, axis=1, keepdims=True)
    lse = m + jnp.log(jnp.sum(jnp.exp(pooled - m), axis=1, keepdims=True))
    out_ref[...] = (pooled - lse)[:, 0:2]


def _pad_edges(src, dst, nepad, trash_base):
    npad = nepad - src.shape[0]
    fill = jnp.arange(npad, dtype=jnp.int32) % 8
    srcp = jnp.concatenate([src, fill]).reshape(-1, JROWS, LANES)
    dstp = jnp.concatenate([dst, trash_base + fill]).reshape(-1, JROWS, LANES)
    return jnp.concatenate([srcp, dstp], axis=1)


def kernel(x, edge_index, batch, Wl1, bl1, Wr1, Wl2, bl2, Wr2, Wl3, bl3, Wr3,
           Wl4, bl4, Wr4):
    f32 = jnp.float32
    src = edge_index[0]
    dst = edge_index[1]
    e2 = _pad_edges(src, dst, EPAD, N)
    pe2 = _pad_edges(jnp.arange(N, dtype=jnp.int32), batch, EPOOL, G)

    ones_col = jnp.ones((N, 1), f32)
    zeros_col = jnp.zeros((N, 1), f32)
    x_pad = jnp.concatenate([x, ones_col, zeros_col], axis=1)

    wl1p = jnp.pad(Wl1.T, ((0, 2), (0, 0)))
    wr1p = jnp.pad(Wr1.T, ((0, 2), (0, 0)))
    wl2g = Wl2.T.reshape(4, 16, H)
    wr2g = Wr2.T.reshape(4, 16, H)
    wl3g = Wl3.T.reshape(4, 16, H)
    wr3g = Wr3.T.reshape(4, 16, H)
    w4cat = jnp.pad(
        jnp.concatenate([Wl4.T, Wr4.T], axis=1), ((0, 0), (0, 12)))
    bl1r = bl1.reshape(1, H)
    bl2r = bl2.reshape(1, H)
    bl3r = bl3.reshape(1, H)
    bl4p = jnp.pad(bl4, (0, 14)).reshape(1, 16)
    sh4 = jnp.asarray(np.eye(16, k=-2, dtype=np.float32))

    wspec = pl.BlockSpec((16, H), lambda i: (0, 0))
    wgspec = pl.BlockSpec((4, 16, H), lambda i: (0, 0, 0))
    bspec = pl.BlockSpec((1, H), lambda i: (0, 0))
    b16spec = pl.BlockSpec((1, 16), lambda i: (0, 0))
    nb16 = pl.BlockSpec((NB, 16), lambda i: (i, 0))
    hgspec = pl.BlockSpec((4, NB, 16), lambda i: (0, i, 0))
    pspec = pl.BlockSpec((1, 2, NB, 16), lambda i: (0, 0, i, 0))
    pgspec = pl.BlockSpec((4, 2, NB, 16), lambda i: (0, 0, i, 0))
    shspec = pl.BlockSpec((16, 16), lambda i: (0, 0))

    p1 = _seg_one(x_pad[None], e2)
    hg1, rc = pl.pallas_call(
        _dense1_body,
        grid=(NGRID,),
        in_specs=[nb16, pspec, wspec, wspec, bspec],
        out_specs=[hgspec, nb16],
        out_shape=[
            jax.ShapeDtypeStruct((4, N, 16), f32),
            jax.ShapeDtypeStruct((N, 16), f32),
        ],
    )(x_pad, p1, wl1p, wr1p, bl1r)

    def conv_mid(hg, wlg, wrg, blr, with_z):
        parts = _seg_four(hg, e2)
        outs = [jax.ShapeDtypeStruct((4, N, 16), f32)]
        out_specs = [hgspec]
        if with_z:
            outs.append(jax.ShapeDtypeStruct((N, 16), f32))
            out_specs.append(nb16)
        body = _dense23z_body if with_z else _dense23n_body
        in_specs = [hgspec, pgspec, nb16, wgspec, wgspec, bspec]
        args = [hg, parts, rc, wlg, wrg, blr]
        if with_z:
            in_specs.append(shspec_w4)
            args.append(w4cat)
        return pl.pallas_call(
            body, grid=(NGRID,), in_specs=in_specs,
            out_specs=out_specs, out_shape=outs,
        )(*args)

    shspec_w4 = pl.BlockSpec((H, 16), lambda i: (0, 0))
    hg2 = conv_mid(hg1, wl2g, wr2g, bl2r, False)[0]
    hg3, z4 = conv_mid(hg2, wl3g, wr3g, bl3r, True)

    p4 = _seg_one(z4[None], e2)
    pp = pl.pallas_call(
        _dense4_body,
        grid=(NGRID,),
        in_specs=[pspec, nb16, nb16, b16spec, shspec],
        out_specs=nb16,
        out_shape=jax.ShapeDtypeStruct((EPOOL, 16), f32),
    )(p4, z4, rc, bl4p, sh4)

    ppart = _seg_pool(pp[None], pe2)
    out = pl.pallas_call(
        _final_body,
        grid=(1,),
        in_specs=[pl.BlockSpec((1, 2, G, 16), lambda i: (0, 0, 0, 0))],
        out_specs=pl.BlockSpec((G, C), lambda i: (0, 0)),
        out_shape=jax.ShapeDtypeStruct((G, C), f32),
    )(ppart)
    return out


def _dense23n_body(hgb_ref, p_ref, rc_ref, wl_ref, wr_ref, b_ref, hg_out):
    acc = jnp.broadcast_to(b_ref[...], (NB, H))
    rc = rc_ref[...]
    for g in range(4):
        mean_g = (p_ref[g, 0] + p_ref[g, 1]) * rc
        acc = acc + jnp.dot(mean_g, wl_ref[g], preferred_element_type=jnp.float32)
        acc = acc + jnp.dot(hgb_ref[g], wr_ref[g], preferred_element_type=jnp.float32)
    h = jnp.maximum(acc, 0.0)
    for g in range(4):
        hg_out[g] = h[:, g * 16:(g + 1) * 16]


def _dense23z_body(hgb_ref, p_ref, rc_ref, wl_ref, wr_ref, b_ref, w4_ref,
                   hg_out, z_out):
    acc = jnp.broadcast_to(b_ref[...], (NB, H))
    rc = rc_ref[...]
    for g in range(4):
        mean_g = (p_ref[g, 0] + p_ref[g, 1]) * rc
        acc = acc + jnp.dot(mean_g, wl_ref[g], preferred_element_type=jnp.float32)
        acc = acc + jnp.dot(hgb_ref[g], wr_ref[g], preferred_element_type=jnp.float32)
    h = jnp.maximum(acc, 0.0)
    for g in range(4):
        hg_out[g] = h[:, g * 16:(g + 1) * 16]
    z_out[...] = jnp.dot(h, w4_ref[...], preferred_element_type=jnp.float32)
